# Initial kernel scaffold; baseline (speedup 1.0000x reference)
import jax
import jax.numpy as jnp
from jax.experimental import pallas as pl


def kernel(x, edge_index, edge_weight, W1, W2):
    raise NotImplementedError("write your pallas kernel here")


# speedup vs baseline: 9.3032x; 9.3032x over previous
import functools

import jax
import jax.numpy as jnp
from jax import lax
from jax.experimental import pallas as pl
from jax.experimental.pallas import tpu as pltpu
from jax.experimental.pallas import tpu_sc as plsc

N = 10000
E = 320000
D = 128
NC = 2
NS = 16
NW = NC * NS
PADN = 10240

E_PER_SC = E // NC
E_PER_TILE = E_PER_SC // NS
B = 80
NCHUNK = E_PER_TILE // B
RPT = PADN // NS
ZR = 128
DZ = PADN // NS

_sc_mesh = plsc.VectorSubcoreMesh(core_axis_name="c", subcore_axis_name="s")
_sc_params = pltpu.CompilerParams(needs_layout_passes=False)


@functools.partial(
    pl.kernel,
    out_type=jax.ShapeDtypeStruct((NC, PADN), jnp.float32),
    mesh=_sc_mesh,
    compiler_params=_sc_params,
    scratch_types=[
        pltpu.VMEM_SHARED((PADN,), jnp.float32),
        pltpu.VMEM((DZ,), jnp.float32),
        pltpu.VMEM((B,), jnp.int32),
        pltpu.VMEM((B,), jnp.float32),
    ],
)
def _deg_kernel(dst_hbm, ew_hbm, out_hbm, dacc, zb, dstb, ewb):
    cid = lax.axis_index("c")
    sid = lax.axis_index("s")

    def zero_body(i, _):
        zb[pl.ds(i * 16, 16)] = jnp.zeros((16,), jnp.float32)
        return 0

    lax.fori_loop(0, DZ // 16, zero_body, 0)
    pltpu.sync_copy(zb, dacc.at[pl.ds(sid * DZ, DZ)])
    plsc.subcore_barrier()

    ebase = cid * E_PER_SC + sid * E_PER_TILE

    def chunk_body(k, _):
        base = ebase + k * B
        pltpu.sync_copy(dst_hbm.at[pl.ds(base, B)], dstb)
        pltpu.sync_copy(ew_hbm.at[pl.ds(base, B)], ewb)
        pltpu.sync_copy(ewb, dacc.at[dstb], add=True)
        return 0

    lax.fori_loop(0, NCHUNK, chunk_body, 0)
    plsc.subcore_barrier()
    pltpu.sync_copy(dacc.at[pl.ds(sid * DZ, DZ)],
                    out_hbm.at[cid, pl.ds(sid * DZ, DZ)])


@functools.partial(
    pl.kernel,
    out_type=jax.ShapeDtypeStruct((NC, PADN, D), jnp.float32),
    mesh=_sc_mesh,
    compiler_params=_sc_params,
    scratch_types=[
        pltpu.VMEM_SHARED((PADN, D), jnp.float32),
        pltpu.VMEM((ZR, D), jnp.float32),
        pltpu.VMEM((B,), jnp.int32),
        pltpu.VMEM((B,), jnp.int32),
        pltpu.VMEM((B,), jnp.float32),
        pltpu.VMEM((B, D), jnp.float32),
        pltpu.SemaphoreType.DMA,
    ],
)
def _edge_kernel(g_hbm, src_hbm, dst_hbm, ew_hbm, agg_hbm,
                 acc, zb, srcb, dstb, ewb, rows, sem):
    cid = lax.axis_index("c")
    sid = lax.axis_index("s")

    def zb_body(i, _):
        for j in range(8):
            zb[i, pl.ds(j * 16, 16)] = jnp.zeros((16,), jnp.float32)
        return 0

    lax.fori_loop(0, ZR, zb_body, 0)
    for k in range(RPT // ZR):
        pltpu.sync_copy(zb, acc.at[pl.ds(sid * RPT + k * ZR, ZR)])
    plsc.subcore_barrier()

    ebase = cid * E_PER_SC + sid * E_PER_TILE

    def chunk_body(ci, _):
        base = ebase + ci * B
        pltpu.sync_copy(src_hbm.at[pl.ds(base, B)], srcb)
        pltpu.sync_copy(dst_hbm.at[pl.ds(base, B)], dstb)
        pltpu.sync_copy(ew_hbm.at[pl.ds(base, B)], ewb)
        pltpu.async_copy(g_hbm.at[srcb], rows, sem).wait()

        def scale_body(e, _):
            w = plsc.load_gather(ewb, [jnp.full((16,), e, jnp.int32)])
            for j in range(8):
                rows[e, pl.ds(j * 16, 16)] = rows[e, pl.ds(j * 16, 16)] * w
            return 0

        lax.fori_loop(0, B, scale_body, 0)
        pltpu.sync_copy(rows, acc.at[dstb], add=True)
        return 0

    lax.fori_loop(0, NCHUNK, chunk_body, 0)
    plsc.subcore_barrier()
    pltpu.sync_copy(acc.at[pl.ds(sid * RPT, RPT)],
                    agg_hbm.at[cid, pl.ds(sid * RPT, RPT)])


def _tc1_body(x_ref, w_ref, pd_ref, g_ref, dis_ref):
    deg = pd_ref[0, :N] + pd_ref[1, :N] + 1.0
    dis = lax.rsqrt(deg)[:, None]
    h = jnp.dot(x_ref[...], w_ref[...], preferred_element_type=jnp.float32)
    g_ref[...] = dis * h
    dis_ref[...] = dis


def _tc2_body(agg_ref, g1_ref, dis_ref, w_ref, g2_ref):
    dis = dis_ref[...]
    out1 = dis * (agg_ref[0, :N] + agg_ref[1, :N] + g1_ref[...])
    x2 = jnp.maximum(out1, 0.0)
    h2 = jnp.dot(x2, w_ref[...], preferred_element_type=jnp.float32)
    g2_ref[...] = dis * h2


def _tc3_body(agg_ref, g2_ref, dis_ref, out_ref):
    out_ref[...] = dis_ref[...] * (agg_ref[0, :N] + agg_ref[1, :N] + g2_ref[...])


_tc1 = pl.pallas_call(
    _tc1_body,
    out_shape=(jax.ShapeDtypeStruct((N, D), jnp.float32),
               jax.ShapeDtypeStruct((N, 1), jnp.float32)),
)

_tc2 = pl.pallas_call(
    _tc2_body,
    out_shape=jax.ShapeDtypeStruct((N, D), jnp.float32),
)

_tc3 = pl.pallas_call(
    _tc3_body,
    out_shape=jax.ShapeDtypeStruct((N, D), jnp.float32),
)


def kernel(x, edge_index, edge_weight, W1, W2):
    src = edge_index[0].astype(jnp.int32)
    dst = edge_index[1].astype(jnp.int32)
    ew = edge_weight.astype(jnp.float32)

    pdeg = _deg_kernel(dst, ew)
    g1, dis = _tc1(x, W1, pdeg)
    agg1 = _edge_kernel(g1, src, dst, ew)
    g2 = _tc2(agg1, g1, dis, W2)
    agg2 = _edge_kernel(g2, src, dst, ew)
    out = _tc3(agg2, g2, dis)
    return out


# speedup vs baseline: 11.4220x; 1.2277x over previous
import functools

import jax
import jax.numpy as jnp
from jax import lax
from jax.experimental import pallas as pl
from jax.experimental.pallas import tpu as pltpu
from jax.experimental.pallas import tpu_sc as plsc

N = 10000
E = 320000
D = 128
NC = 2
NS = 16
NW = NC * NS
PADN = 10240

E_PER_TILE = E // NW
B = 80
NCHUNK = E_PER_TILE // B
RPT = PADN // NS
ZR = 16
DEGW = 8

_sc_mesh = plsc.VectorSubcoreMesh(core_axis_name="c", subcore_axis_name="s")
_sc_params = pltpu.CompilerParams(needs_layout_passes=False)


@functools.partial(
    pl.kernel,
    out_type=jax.ShapeDtypeStruct((NC, PADN), jnp.float32),
    mesh=_sc_mesh,
    compiler_params=_sc_params,
    scratch_types=[
        pltpu.VMEM_SHARED((PADN,), jnp.float32),
        pltpu.VMEM((PADN // NS,), jnp.float32),
        pltpu.VMEM((2, B), jnp.int32),
        pltpu.VMEM((2, B), jnp.float32),
        pltpu.SemaphoreType.DMA,
        pltpu.SemaphoreType.DMA,
        pltpu.SemaphoreType.DMA,
        pltpu.SemaphoreType.DMA,
        pltpu.SemaphoreType.DMA,
        pltpu.SemaphoreType.DMA,
    ],
)
def _deg_kernel(dst_hbm, ew_hbm, out_hbm, dacc, zb, dstb2, ewb2,
                dsem0, dsem1, esem0, esem1, ssem0, ssem1):
    cid = lax.axis_index("c")
    sid = lax.axis_index("s")
    wid = cid * NS + sid
    ebase = wid * E_PER_TILE
    dz = PADN // NS
    dsem = (dsem0, dsem1)
    esem = (esem0, esem1)
    ssem = (ssem0, ssem1)

    def zero_body(i, _):
        zb[pl.ds(i * 16, 16)] = jnp.zeros((16,), jnp.float32)
        return 0

    lax.fori_loop(0, dz // 16, zero_body, 0)
    pltpu.sync_copy(zb, dacc.at[pl.ds(sid * dz, dz)])
    plsc.subcore_barrier()

    for b in range(2):
        pltpu.async_copy(dst_hbm.at[pl.ds(ebase + b * B, B)], dstb2.at[b],
                         dsem[b])
        pltpu.async_copy(ew_hbm.at[pl.ds(ebase + b * B, B)], ewb2.at[b],
                         esem[b])

    def deg_chunk(kd, b, prefetch):
        pltpu.make_async_copy(dst_hbm.at[pl.ds(ebase + kd * B, B)],
                              dstb2.at[b], dsem[b]).wait()
        pltpu.make_async_copy(ew_hbm.at[pl.ds(ebase + kd * B, B)],
                              ewb2.at[b], esem[b]).wait()
        pltpu.sync_copy(ewb2.at[b], dacc.at[dstb2.at[b]], add=True)
        if prefetch:
            @pl.when(kd + 2 < NCHUNK)
            def _next():
                pltpu.async_copy(dst_hbm.at[pl.ds(ebase + (kd + 2) * B, B)],
                                 dstb2.at[b], dsem[b])
                pltpu.async_copy(ew_hbm.at[pl.ds(ebase + (kd + 2) * B, B)],
                                 ewb2.at[b], esem[b])

    def deg_pair(j, _):
        for b in range(2):
            deg_chunk(2 * j + b, b, True)
        return 0

    lax.fori_loop(0, NCHUNK // 2, deg_pair, 0)
    deg_chunk(NCHUNK - 1, 0, False)
    plsc.subcore_barrier()
    pltpu.sync_copy(dacc.at[pl.ds(sid * dz, dz)],
                    out_hbm.at[cid, pl.ds(sid * dz, dz)])


@functools.partial(
    pl.kernel,
    out_type=jax.ShapeDtypeStruct((NC, PADN, D), jnp.float32),
    mesh=_sc_mesh,
    compiler_params=_sc_params,
    scratch_types=[
        pltpu.VMEM_SHARED((PADN, D), jnp.float32),
        pltpu.VMEM((ZR, D), jnp.float32),
        pltpu.VMEM((2, B), jnp.int32),
        pltpu.VMEM((2, B), jnp.int32),
        pltpu.VMEM((2, B), jnp.float32),
        pltpu.VMEM((B, D), jnp.float32),
        pltpu.VMEM((B, D), jnp.float32),
        pltpu.VMEM((B, D), jnp.float32),
        pltpu.VMEM((B, D), jnp.float32),
        pltpu.SemaphoreType.DMA,
        pltpu.SemaphoreType.DMA,
        pltpu.SemaphoreType.DMA,
        pltpu.SemaphoreType.DMA,
        pltpu.SemaphoreType.DMA,
        pltpu.SemaphoreType.DMA,
        pltpu.SemaphoreType.DMA,
        pltpu.SemaphoreType.DMA,
        pltpu.SemaphoreType.DMA,
        pltpu.SemaphoreType.DMA,
    ],
)
def _edge_kernel(g_hbm, src_hbm, dst_hbm, ew_hbm, agg_hbm,
                 acc, zb, srcb2, dstb2, ewb2, rows0, rows1, srows0, srows1,
                 gsem0, gsem1, ssem0, ssem1, csem0, csem1, dsem0, dsem1,
                 esem0, esem1):
    cid = lax.axis_index("c")
    sid = lax.axis_index("s")
    wid = cid * NS + sid
    ebase = wid * E_PER_TILE
    rows = (rows0, rows1)
    srows = (srows0, srows1)
    gsem = (gsem0, gsem1)
    ssem = (ssem0, ssem1)
    csem = (csem0, csem1)
    dsem = (dsem0, dsem1)
    esem = (esem0, esem1)

    def zb_body(i, _):
        for j in range(8):
            zb[i, pl.ds(j * 16, 16)] = jnp.zeros((16,), jnp.float32)
        return 0

    lax.fori_loop(0, ZR, zb_body, 0)
    for k in range(RPT // ZR):
        pltpu.sync_copy(zb, acc.at[pl.ds(sid * RPT + k * ZR, ZR)])
    plsc.subcore_barrier()

    for b in range(2):
        pltpu.async_copy(src_hbm.at[pl.ds(ebase + b * B, B)], srcb2.at[b], csem[b])
        pltpu.async_copy(ew_hbm.at[pl.ds(ebase + b * B, B)], ewb2.at[b], esem[b])
    for b in range(2):
        pltpu.make_async_copy(src_hbm.at[pl.ds(ebase + b * B, B)], srcb2.at[b],
                              csem[b]).wait()
        pltpu.async_copy(g_hbm.at[srcb2.at[b]], rows[b], gsem[b])

    def do_chunk(kd, b, first, prefetch):
        pltpu.make_async_copy(g_hbm.at[srcb2.at[b]], rows[b],
                              gsem[b]).wait()
        if prefetch:
            @pl.when(kd + 2 < NCHUNK)
            def _fire_src():
                pltpu.async_copy(src_hbm.at[pl.ds(ebase + (kd + 2) * B, B)], srcb2.at[b],
                                 csem[b])

        @pl.when(jnp.logical_not(first))
        def _wait_scatter():
            pltpu.make_async_copy(srows[b], acc.at[dstb2.at[b]],
                                  ssem[b]).wait()
        pltpu.async_copy(dst_hbm.at[pl.ds(ebase + kd * B, B)], dstb2.at[b], dsem[b])

        pltpu.make_async_copy(ew_hbm.at[pl.ds(ebase + kd * B, B)], ewb2.at[b],
                              esem[b]).wait()

        def scale_body(e, _):
            w = plsc.load_gather(ewb2.at[b], [jnp.full((16,), e, jnp.int32)])
            for jj in range(8):
                srows[b][e, pl.ds(jj * 16, 16)] = (
                    rows[b][e, pl.ds(jj * 16, 16)] * w)
            return 0

        lax.fori_loop(0, B, scale_body, 0)
        if prefetch:
            @pl.when(kd + 2 < NCHUNK)
            def _fire_ew():
                pltpu.async_copy(ew_hbm.at[pl.ds(ebase + (kd + 2) * B, B)], ewb2.at[b],
                                 esem[b])

        pltpu.make_async_copy(dst_hbm.at[pl.ds(ebase + kd * B, B)], dstb2.at[b],
                              dsem[b]).wait()
        pltpu.async_copy(srows[b], acc.at[dstb2.at[b]], ssem[b], add=True)
        if prefetch:
            @pl.when(kd + 2 < NCHUNK)
            def _fire_gather():
                pltpu.make_async_copy(src_hbm.at[pl.ds(ebase + (kd + 2) * B, B)], srcb2.at[b],
                                      csem[b]).wait()
                pltpu.async_copy(g_hbm.at[srcb2.at[b]], rows[b], gsem[b])

    def pair_body(j, _):
        for b in range(2):
            do_chunk(2 * j + b, b, j == 0, True)
        return 0

    lax.fori_loop(0, NCHUNK // 2, pair_body, 0)
    do_chunk(NCHUNK - 1, 0, False, False)
    pltpu.make_async_copy(srows[0], acc.at[dstb2.at[0]], ssem[0]).wait()
    pltpu.make_async_copy(srows[1], acc.at[dstb2.at[1]], ssem[1]).wait()
    plsc.subcore_barrier()

    for k in range(RPT // 64):
        off = sid * RPT + k * 64
        pltpu.sync_copy(acc.at[pl.ds(off, 64)], srows0.at[pl.ds(0, 64)])
        pltpu.sync_copy(srows0.at[pl.ds(0, 64)],
                        agg_hbm.at[cid, pl.ds(off, 64)])


def _tc1_body(x_ref, w_ref, pd_ref, g_ref, dis_ref):
    deg = pd_ref[0, :N] + pd_ref[1, :N] + 1.0
    dis = lax.rsqrt(deg)[:, None]
    h = jnp.dot(x_ref[...], w_ref[...], preferred_element_type=jnp.float32)
    g_ref[...] = dis * h
    dis_ref[...] = dis


def _tc2_body(agg_ref, g1_ref, dis_ref, w_ref, g2_ref):
    dis = dis_ref[...]
    out1 = dis * (agg_ref[0, :N] + agg_ref[1, :N] + g1_ref[...])
    x2 = jnp.maximum(out1, 0.0)
    h2 = jnp.dot(x2, w_ref[...], preferred_element_type=jnp.float32)
    g2_ref[...] = dis * h2


def _tc3_body(agg_ref, g2_ref, dis_ref, out_ref):
    out_ref[...] = dis_ref[...] * (agg_ref[0, :N] + agg_ref[1, :N] + g2_ref[...])


_tc1 = pl.pallas_call(
    _tc1_body,
    out_shape=(jax.ShapeDtypeStruct((N, D), jnp.float32),
               jax.ShapeDtypeStruct((N, 1), jnp.float32)),
)

_tc2 = pl.pallas_call(
    _tc2_body,
    out_shape=jax.ShapeDtypeStruct((N, D), jnp.float32),
)

_tc3 = pl.pallas_call(
    _tc3_body,
    out_shape=jax.ShapeDtypeStruct((N, D), jnp.float32),
)


def kernel(x, edge_index, edge_weight, W1, W2):
    src = edge_index[0].astype(jnp.int32)
    dst = edge_index[1].astype(jnp.int32)
    ew = edge_weight.astype(jnp.float32)

    pdeg = _deg_kernel(dst, ew)
    g1, dis = _tc1(x, W1, pdeg)
    agg1 = _edge_kernel(g1, src, dst, ew)
    g2 = _tc2(agg1, g1, dis, W2)
    agg2 = _edge_kernel(g2, src, dst, ew)
    out = _tc3(agg2, g2, dis)
    return out
